# baseline (speedup 1.0000x reference)
import jax
import jax.numpy as jnp
from jax import lax
from jax.experimental import pallas as pl
from jax.experimental.pallas import tpu as pltpu
from jax.experimental.pallas import tpu_sc as plsc

B, C, H, W = 16, 3, 512, 512
IMGPX = H * W
CHUNK = 16384
LANES = 16
NBINS = 256
UNROLL = 4
ROWS = 4
RB = H // ROWS
NGRP = 2
GB = B // NGRP
NTILES = 32
NCORES = 2
NSUB = 16
TPI = NTILES // GB
IPC = NSUB // TPI
PARTPX = IMGPX // TPI
NCHUNK = PARTPX // CHUNK


def _codes_body(img_ref, code_ref):
    r = jnp.clip(img_ref[0, 0], 0.0, 1.0)
    g = jnp.clip(img_ref[0, 1], 0.0, 1.0)
    b = jnp.clip(img_ref[0, 2], 0.0, 1.0)
    y = 0.299 * r + 0.587 * g + 0.114 * b
    im255 = y * 255.0
    bin_f = jnp.clip(jnp.floor(im255 * (256.0 / 255.0)), 0.0, 255.0)
    idx_f = jnp.clip(im255, 0.0, 255.0)
    codes = bin_f.astype(jnp.int32) * 256 + idx_f.astype(jnp.int32)
    code_ref[...] = codes.reshape(RB * W)


def _finish_body(img_ref, yeq_ref, step_ref, out_ref):
    r = jnp.clip(img_ref[0, 0], 0.0, 1.0)
    g = jnp.clip(img_ref[0, 1], 0.0, 1.0)
    b = jnp.clip(img_ref[0, 2], 0.0, 1.0)
    y = 0.299 * r + 0.587 * g + 0.114 * b
    u = -0.147 * r - 0.289 * g + 0.436 * b
    v = 0.615 * r - 0.515 * g - 0.100 * b
    cond = step_ref[0, 0:1, 0:1] == 0.0
    yf = jnp.where(cond, y * 255.0, yeq_ref[...].reshape(RB, W)) / 255.0
    out_ref[0, 0] = yf + 1.14 * v
    out_ref[0, 1] = yf - 0.396 * u - 0.581 * v
    out_ref[0, 2] = yf + 2.029 * u


def _sc_body(codes_hbm, yeq_hbm, step_hbm,
             buf0, buf1, ob0, ob1, histf, hsum, phist, lut, stepbuf, shared,
             si0, si1, so0, so1):
    cid = lax.axis_index("c")
    sid = lax.axis_index("s")
    limg = cid * IPC + sid // TPI
    part = sid % TPI
    base = limg * IMGPX + part * PARTPX

    bufs = (buf0, buf1)
    obufs = (ob0, ob1)
    isems = (si0, si1)
    osems = (so0, so1)

    lane = lax.iota(jnp.int32, LANES)
    lane256 = lane * NBINS
    ones = jnp.full((LANES,), 1.0, jnp.float32)
    zeros16 = jnp.zeros((LANES,), jnp.float32)

    @plsc.parallel_loop(0, (LANES * NBINS) // LANES, 1, unroll=8)
    def _(i):
        histf[pl.ds(i * LANES, LANES)] = zeros16

    def copy_in(k):
        return pltpu.async_copy(
            codes_hbm.at[pl.ds(base + k * CHUNK, CHUNK)],
            bufs[k % 2], isems[k % 2])

    hnd = copy_in(0)
    for k in range(NCHUNK):
        nxt = copy_in(k + 1) if k + 1 < NCHUNK else None
        hnd.wait()
        bbuf = bufs[k % 2]

        @plsc.parallel_loop(0, CHUNK // LANES, 1, unroll=UNROLL)
        def _(i, bbuf=bbuf):
            w = bbuf[pl.ds(i * LANES, LANES)]
            plsc.addupdate_scatter(histf, [lane256 + (w >> 8)], ones)
        hnd = nxt

    for j in range(NBINS // LANES):
        acc = zeros16
        for l in range(LANES):
            acc = acc + histf[pl.ds(l * NBINS + j * LANES, LANES)]
        hsum[pl.ds(j * LANES, LANES)] = acc

    pltpu.sync_copy(hsum, shared.at[sid])
    plsc.subcore_barrier()
    row0 = sid - part
    pltpu.sync_copy(shared.at[row0], hsum)
    for t in range(1, TPI):
        pltpu.sync_copy(shared.at[row0 + t], phist)
        for j in range(NBINS // LANES):
            sl = pl.ds(j * LANES, LANES)
            hsum[sl] = hsum[sl] + phist[sl]

    m = jnp.full((LANES,), -1, jnp.int32)
    for j in range(NBINS // LANES):
        hj = hsum[pl.ds(j * LANES, LANES)]
        m = jnp.where(hj > 0.0, lane + j * LANES, m)
    last_idx = jnp.max(m)
    last_nz = plsc.load_gather(hsum, [jnp.full((LANES,), last_idx, jnp.int32)])
    step = ((float(IMGPX) - last_nz) / 255.0).astype(jnp.int32).astype(jnp.float32)
    safe_step = jnp.maximum(step, 1.0)
    off = (step * 0.5).astype(jnp.int32).astype(jnp.float32)

    carry = zeros16
    for j in range(NBINS // LANES):
        sl = pl.ds(j * LANES, LANES)
        hj = hsum[sl]
        inc = plsc.cumsum(hj)
        excl = inc - hj + carry
        t = (excl + off) / safe_step
        lut[sl] = jnp.clip(t.astype(jnp.int32), 0, 255).astype(jnp.float32)
        carry = carry + jnp.sum(hj)

    stepbuf[...] = step

    @pl.when(part == 0)
    def _():
        pltpu.sync_copy(stepbuf, step_hbm.at[pl.ds(limg * LANES, LANES)])

    hnd = copy_in(0)
    onds = [None] * NCHUNK
    for k in range(NCHUNK):
        nxt = copy_in(k + 1) if k + 1 < NCHUNK else None
        hnd.wait()
        if k >= 2:
            onds[k - 2].wait()
        bbuf = bufs[k % 2]
        obuf = obufs[k % 2]

        @plsc.parallel_loop(0, CHUNK // LANES, 1, unroll=UNROLL)
        def _(i, bbuf=bbuf, obuf=obuf):
            sl = pl.ds(i * LANES, LANES)
            w = bbuf[sl]
            obuf[sl] = plsc.load_gather(lut, [w & 255])
        onds[k] = pltpu.async_copy(
            obuf, yeq_hbm.at[pl.ds(base + k * CHUNK, CHUNK)], osems[k % 2])
        hnd = nxt
    if NCHUNK >= 2:
        onds[NCHUNK - 2].wait()
    onds[NCHUNK - 1].wait()


def _sc_equalize(codes_flat):
    mesh = plsc.VectorSubcoreMesh(core_axis_name="c", subcore_axis_name="s")
    fn = pl.kernel(
        _sc_body,
        out_type=(jax.ShapeDtypeStruct((GB * IMGPX,), jnp.float32),
                  jax.ShapeDtypeStruct((GB * LANES,), jnp.float32)),
        mesh=mesh,
        compiler_params=pltpu.CompilerParams(needs_layout_passes=False),
        scratch_types=[
            pltpu.VMEM((CHUNK,), jnp.int32),
            pltpu.VMEM((CHUNK,), jnp.int32),
            pltpu.VMEM((CHUNK,), jnp.float32),
            pltpu.VMEM((CHUNK,), jnp.float32),
            pltpu.VMEM((LANES * NBINS,), jnp.float32),
            pltpu.VMEM((NBINS,), jnp.float32),
            pltpu.VMEM((NBINS,), jnp.float32),
            pltpu.VMEM((NBINS,), jnp.float32),
            pltpu.VMEM((LANES,), jnp.float32),
            pltpu.VMEM_SHARED((NSUB, NBINS), jnp.float32),
            pltpu.SemaphoreType.DMA,
            pltpu.SemaphoreType.DMA,
            pltpu.SemaphoreType.DMA,
            pltpu.SemaphoreType.DMA,
        ],
    )
    return fn(codes_flat)


def kernel(img):
    outs = []
    for grp in range(NGRP):
        goff = grp * GB
        codes = pl.pallas_call(
            _codes_body,
            grid=(GB, ROWS),
            in_specs=[pl.BlockSpec((1, C, RB, W),
                                   lambda i, j, goff=goff: (goff + i, 0, j, 0))],
            out_specs=pl.BlockSpec((RB * W,), lambda i, j: (i * ROWS + j,)),
            out_shape=jax.ShapeDtypeStruct((GB * IMGPX,), jnp.int32),
        )(img)

        yeq_flat, step_raw = _sc_equalize(codes)
        step128 = jnp.tile(step_raw.reshape(GB, LANES), (1, 64)).reshape(GB, 8, 128)

        outs.append(pl.pallas_call(
            _finish_body,
            grid=(GB, ROWS),
            in_specs=[
                pl.BlockSpec((1, C, RB, W),
                             lambda i, j, goff=goff: (goff + i, 0, j, 0)),
                pl.BlockSpec((RB * W,), lambda i, j: (i * ROWS + j,)),
                pl.BlockSpec((1, 8, 128), lambda i, j: (i, 0, 0)),
            ],
            out_specs=pl.BlockSpec((1, C, RB, W), lambda i, j: (i, 0, j, 0)),
            out_shape=jax.ShapeDtypeStruct((GB, C, H, W), jnp.float32),
        )(img, yeq_flat, step128))
    return jnp.concatenate(outs, axis=0)


# speedup vs baseline: 1.3858x; 1.3858x over previous
import jax
import jax.numpy as jnp
from jax import lax
from jax.experimental import pallas as pl
from jax.experimental.pallas import tpu as pltpu
from jax.experimental.pallas import tpu_sc as plsc

B, C, H, W = 16, 3, 512, 512
IMGPX = H * W
CHUNK = 16384
LANES = 16
NBINS = 256
UNROLL = 4
ROWS = 2
RB = H // ROWS
NGRP = 1
STEPW = 1024
GB = B // NGRP
NTILES = 32
NCORES = 2
NSUB = 16
TPI = NTILES // GB
IPC = NSUB // TPI
PARTPX = IMGPX // TPI
NCHUNK = PARTPX // CHUNK


def _codes_body(img_ref, code_ref):
    r = jnp.clip(img_ref[0, 0], 0.0, 1.0)
    g = jnp.clip(img_ref[0, 1], 0.0, 1.0)
    b = jnp.clip(img_ref[0, 2], 0.0, 1.0)
    y = 0.299 * r + 0.587 * g + 0.114 * b
    im255 = y * 255.0
    bin_f = jnp.clip(jnp.floor(im255 * (256.0 / 255.0)), 0.0, 255.0)
    idx_f = jnp.clip(im255, 0.0, 255.0)
    codes = bin_f.astype(jnp.int32) * 256 + idx_f.astype(jnp.int32)
    code_ref[...] = codes.reshape(RB * W)


def _finish_body(img_ref, yeq_ref, step_ref, out_ref):
    r = jnp.clip(img_ref[0, 0], 0.0, 1.0)
    g = jnp.clip(img_ref[0, 1], 0.0, 1.0)
    b = jnp.clip(img_ref[0, 2], 0.0, 1.0)
    y = 0.299 * r + 0.587 * g + 0.114 * b
    u = -0.147 * r - 0.289 * g + 0.436 * b
    v = 0.615 * r - 0.515 * g - 0.100 * b
    cond = step_ref[...].reshape(8, 128)[0:1, 0:1] == 0.0
    yf = jnp.where(cond, y * 255.0, yeq_ref[...].reshape(RB, W)) / 255.0
    out_ref[0, 0] = yf + 1.14 * v
    out_ref[0, 1] = yf - 0.396 * u - 0.581 * v
    out_ref[0, 2] = yf + 2.029 * u


def _sc_body(codes_hbm, yeq_hbm, step_hbm,
             buf0, buf1, ob0, ob1, histf, hsum, phist, lut, stepbuf, shared,
             si0, si1, so0, so1):
    cid = lax.axis_index("c")
    sid = lax.axis_index("s")
    limg = cid * IPC + sid // TPI
    part = sid % TPI
    base = limg * IMGPX + part * PARTPX

    bufs = (buf0, buf1)
    obufs = (ob0, ob1)
    isems = (si0, si1)
    osems = (so0, so1)

    lane = lax.iota(jnp.int32, LANES)
    lane256 = lane * NBINS
    ones = jnp.full((LANES,), 1.0, jnp.float32)
    zeros16 = jnp.zeros((LANES,), jnp.float32)

    @plsc.parallel_loop(0, (LANES * NBINS) // LANES, 1, unroll=8)
    def _(i):
        histf[pl.ds(i * LANES, LANES)] = zeros16

    def copy_in(k):
        return pltpu.async_copy(
            codes_hbm.at[pl.ds(base + k * CHUNK, CHUNK)],
            bufs[k % 2], isems[k % 2])

    hnd = copy_in(0)
    for k in range(NCHUNK):
        nxt = copy_in(k + 1) if k + 1 < NCHUNK else None
        hnd.wait()
        bbuf = bufs[k % 2]

        @plsc.parallel_loop(0, CHUNK // LANES, 1, unroll=UNROLL)
        def _(i, bbuf=bbuf):
            w = bbuf[pl.ds(i * LANES, LANES)]
            plsc.addupdate_scatter(histf, [lane256 + (w >> 8)], ones)
        hnd = nxt

    for j in range(NBINS // LANES):
        acc = zeros16
        for l in range(LANES):
            acc = acc + histf[pl.ds(l * NBINS + j * LANES, LANES)]
        hsum[pl.ds(j * LANES, LANES)] = acc

    pltpu.sync_copy(hsum, shared.at[sid])
    plsc.subcore_barrier()
    row0 = sid - part
    pltpu.sync_copy(shared.at[row0], hsum)
    for t in range(1, TPI):
        pltpu.sync_copy(shared.at[row0 + t], phist)
        for j in range(NBINS // LANES):
            sl = pl.ds(j * LANES, LANES)
            hsum[sl] = hsum[sl] + phist[sl]

    m = jnp.full((LANES,), -1, jnp.int32)
    for j in range(NBINS // LANES):
        hj = hsum[pl.ds(j * LANES, LANES)]
        m = jnp.where(hj > 0.0, lane + j * LANES, m)
    last_idx = jnp.max(m)
    last_nz = plsc.load_gather(hsum, [jnp.full((LANES,), last_idx, jnp.int32)])
    step = ((float(IMGPX) - last_nz) / 255.0).astype(jnp.int32).astype(jnp.float32)
    safe_step = jnp.maximum(step, 1.0)
    off = (step * 0.5).astype(jnp.int32).astype(jnp.float32)

    carry = zeros16
    for j in range(NBINS // LANES):
        sl = pl.ds(j * LANES, LANES)
        hj = hsum[sl]
        inc = plsc.cumsum(hj)
        excl = inc - hj + carry
        t = (excl + off) / safe_step
        lut[sl] = jnp.clip(t.astype(jnp.int32), 0, 255).astype(jnp.float32)
        carry = carry + jnp.sum(hj)

    @plsc.parallel_loop(0, STEPW // LANES, 1, unroll=8)
    def _(i):
        stepbuf[pl.ds(i * LANES, LANES)] = step

    @pl.when(part == 0)
    def _():
        pltpu.sync_copy(stepbuf, step_hbm.at[pl.ds(limg * STEPW, STEPW)])

    hnd = copy_in(0)
    onds = [None] * NCHUNK
    for k in range(NCHUNK):
        nxt = copy_in(k + 1) if k + 1 < NCHUNK else None
        hnd.wait()
        if k >= 2:
            onds[k - 2].wait()
        bbuf = bufs[k % 2]
        obuf = obufs[k % 2]

        @plsc.parallel_loop(0, CHUNK // LANES, 1, unroll=UNROLL)
        def _(i, bbuf=bbuf, obuf=obuf):
            sl = pl.ds(i * LANES, LANES)
            w = bbuf[sl]
            obuf[sl] = plsc.load_gather(lut, [w & 255])
        onds[k] = pltpu.async_copy(
            obuf, yeq_hbm.at[pl.ds(base + k * CHUNK, CHUNK)], osems[k % 2])
        hnd = nxt
    if NCHUNK >= 2:
        onds[NCHUNK - 2].wait()
    onds[NCHUNK - 1].wait()


def _sc_equalize(codes_flat):
    mesh = plsc.VectorSubcoreMesh(core_axis_name="c", subcore_axis_name="s")
    fn = pl.kernel(
        _sc_body,
        out_type=(jax.ShapeDtypeStruct((GB * IMGPX,), jnp.float32),
                  jax.ShapeDtypeStruct((GB * STEPW,), jnp.float32)),
        mesh=mesh,
        compiler_params=pltpu.CompilerParams(needs_layout_passes=False),
        scratch_types=[
            pltpu.VMEM((CHUNK,), jnp.int32),
            pltpu.VMEM((CHUNK,), jnp.int32),
            pltpu.VMEM((CHUNK,), jnp.float32),
            pltpu.VMEM((CHUNK,), jnp.float32),
            pltpu.VMEM((LANES * NBINS,), jnp.float32),
            pltpu.VMEM((NBINS,), jnp.float32),
            pltpu.VMEM((NBINS,), jnp.float32),
            pltpu.VMEM((NBINS,), jnp.float32),
            pltpu.VMEM((STEPW,), jnp.float32),
            pltpu.VMEM_SHARED((NSUB, NBINS), jnp.float32),
            pltpu.SemaphoreType.DMA,
            pltpu.SemaphoreType.DMA,
            pltpu.SemaphoreType.DMA,
            pltpu.SemaphoreType.DMA,
        ],
    )
    return fn(codes_flat)


def kernel(img):
    outs = []
    for grp in range(NGRP):
        goff = grp * GB
        codes = pl.pallas_call(
            _codes_body,
            grid=(GB, ROWS),
            in_specs=[pl.BlockSpec((1, C, RB, W),
                                   lambda i, j, goff=goff: (goff + i, 0, j, 0))],
            out_specs=pl.BlockSpec((RB * W,), lambda i, j: (i * ROWS + j,)),
            out_shape=jax.ShapeDtypeStruct((GB * IMGPX,), jnp.int32),
        )(img)

        yeq_flat, step_raw = _sc_equalize(codes)

        outs.append(pl.pallas_call(
            _finish_body,
            grid=(GB, ROWS),
            in_specs=[
                pl.BlockSpec((1, C, RB, W),
                             lambda i, j, goff=goff: (goff + i, 0, j, 0)),
                pl.BlockSpec((RB * W,), lambda i, j: (i * ROWS + j,)),
                pl.BlockSpec((STEPW,), lambda i, j: (i,)),
            ],
            out_specs=pl.BlockSpec((1, C, RB, W), lambda i, j: (i, 0, j, 0)),
            out_shape=jax.ShapeDtypeStruct((GB, C, H, W), jnp.float32),
        )(img, yeq_flat, step_raw))
    if NGRP == 1:
        return outs[0].reshape(B, C, H, W)
    return jnp.concatenate(outs, axis=0)
